# baseline (speedup 1.0000x reference)
import functools

import jax
import jax.numpy as jnp
import numpy as np
from jax import lax
from jax.experimental import pallas as pl
from jax.experimental.pallas import tpu as pltpu
from jax.experimental.pallas import tpu_sc as plsc

EMBED_DIM = 64
BATCH = 4
SEQ = 4096
B = BATCH * SEQ
VOCAB = 1000000
NW = 32
RANGE = VOCAB // NW
SLABW = 256
NSLAB = 124
LCAP = 1024
SUBN = 8
SUBCAP = 256
GCAP = 256
LASTC = (VOCAB // SLABW) * SLABW - SLABW
TAIL0 = LASTC + SLABW
DUMP = B
OUTPAD = B + 128


def _positional_encoding_np(position, d_model):
    pos = np.arange(position)[:, np.newaxis].astype(np.float64)
    i = np.arange(d_model)[np.newaxis, :].astype(np.float64)
    angle_rates = 1.0 / np.power(10000, 2 * (i // 2) / np.float32(d_model))
    angle_rads = pos * angle_rates
    angle_rads[:, 0::2] = np.sin(angle_rads[:, 0::2])
    angle_rads[:, 1::2] = np.cos(angle_rads[:, 1::2])
    return angle_rads.astype(np.float32)


_POS_NP = _positional_encoding_np(SEQ, EMBED_DIM)

_mesh = plsc.VectorSubcoreMesh(core_axis_name="c", subcore_axis_name="s")
_IOTA = functools.partial(lax.broadcasted_iota, jnp.int32, (16,))


def _scan_body(tT, xr, tailp, rows1, tlist2, x_v, slab0, slab1, rows_v, idsb,
               tposb, subid, sublp, gcol, glp, sem0, sem1):
    wid = lax.axis_index("s") * 2 + lax.axis_index("c")
    lo = wid * RANGE
    c0 = (lo // SLABW) * SLABW
    iota = _IOTA(0)

    def slab_ds(start):
        return pl.ds(pl.multiple_of(jnp.minimum(start, LASTC), SLABW), SLABW)

    pltpu.sync_copy(xr, x_v)

    def init(v, _):
        tposb[pl.ds(v * 16, 16)] = jnp.zeros((16,), jnp.int32) + DUMP
        return 0

    lax.fori_loop(0, LCAP // 16, init, 0)

    pltpu.async_copy(tT.at[:, slab_ds(c0)], slab0, sem0)
    pltpu.async_copy(tT.at[:, slab_ds(c0 + SLABW)], slab1, sem1)

    def pref(v, n):
        ids = x_v[pl.ds(v * 16, 16)]
        m = (ids >= lo) & (ids < lo + RANGE)
        cnt = jnp.sum(jnp.where(m, 1, 0))
        nw = jnp.minimum(n, LCAP - 16)
        plsc.store_compressed(idsb.at[pl.ds(nw, 16)], ids, mask=m)
        plsc.store_compressed(tposb.at[pl.ds(nw, 16)], v * 16 + iota, mask=m)
        return n + cnt

    n = lax.fori_loop(0, 0, pref, jnp.zeros((), jnp.int32))

    def binstep(v, ks):
        ids = idsb[pl.ds(v * 16, 16)]
        lp = v * 16 + iota
        valid = lp < n
        sb = (ids - lo) >> 12
        out = []
        for s in range(SUBN):
            m = valid & (sb == s)
            cnt = jnp.sum(jnp.where(m, 1, 0))
            kw = jnp.minimum(ks[s], SUBCAP - 16)
            plsc.store_compressed(subid.at[pl.ds(s * SUBCAP + kw, 16)], ids, mask=m)
            plsc.store_compressed(sublp.at[pl.ds(s * SUBCAP + kw, 16)], lp, mask=m)
            out.append(ks[s] + cnt)
        return tuple(out)

    z = jnp.zeros((), jnp.int32)
    ks = lax.fori_loop(0, LCAP // 16, binstep, (z,) * SUBN)
    ns_vec = jnp.zeros((16,), jnp.int32)
    for s in range(SUBN):
        ns_vec = jnp.where(iota == s, ks[s], ns_vec)

    def sub_scan(s, start, k):
        ksub = jnp.sum(jnp.where(iota == s, ns_vec, 0))

        def scan(v, k):
            ids = subid[pl.ds(s * SUBCAP + v * 16, 16)]
            lp = sublp[pl.ds(s * SUBCAP + v * 16, 16)]
            valid = (v * 16 + iota) < ksub
            m = valid & (ids >= start) & (ids < start + SLABW)
            cnt = jnp.sum(jnp.where(m, 1, 0))
            kw = jnp.minimum(k, GCAP - 16)
            plsc.store_compressed(gcol.at[pl.ds(kw, 16)], ids - start, mask=m)
            plsc.store_compressed(glp.at[pl.ds(kw, 16)], lp, mask=m)
            return k + cnt

        return lax.fori_loop(0, (ksub + 15) >> 4, scan, k)

    def process(buf, start):
        sa = jnp.clip((start - lo) >> 12, 0, SUBN - 1)
        sb_ = jnp.clip((start + SLABW - 1 - lo) >> 12, 0, SUBN - 1)
        k = sub_scan(sa, start, jnp.zeros((), jnp.int32))
        k = sub_scan(jnp.minimum(jnp.maximum(sb_, sa + 1), SUBN - 1), start, k)

        def extract(g, _):
            msk = (g * 16 + iota) < k
            colsv = gcol[pl.ds(g * 16, 16)]
            lpv = glp[pl.ds(g * 16, 16)]
            for dd in range(EMBED_DIM):
                vals = plsc.load_gather(
                    buf, [jnp.zeros((16,), jnp.int32) + dd, colsv], mask=msk)
                plsc.store_scatter(
                    rows_v, [lpv >> 1, (lpv & 1) * 64 + dd], vals, mask=msk)
            return 0

        lax.fori_loop(0, (k + 15) >> 4, extract, 0)

    def slabloop(p, _):
        for b, (buf, sem) in enumerate(((slab0, sem0), (slab1, sem1))):
            ci = p * 2 + b
            start = jnp.minimum(c0 + ci * SLABW, LASTC)
            pltpu.make_async_copy(tT.at[:, pl.ds(0, SLABW)], buf, sem).wait()
            process(buf, start)
            pltpu.async_copy(tT.at[:, slab_ds(c0 + (ci + 2) * SLABW)], buf, sem)
        return 0

    lax.fori_loop(0, NSLAB // 2, slabloop, 0)
    pltpu.make_async_copy(tT.at[:, pl.ds(0, SLABW)], slab0, sem0).wait()
    pltpu.make_async_copy(tT.at[:, pl.ds(0, SLABW)], slab1, sem1).wait()

    @pl.when(wid == NW - 1)
    def _():
        pltpu.sync_copy(tailp, slab0)
        process(slab0, jnp.zeros((), jnp.int32) + TAIL0)

    pltpu.sync_copy(rows_v, rows1.at[pl.ds(wid * (LCAP // 2), LCAP // 2)])
    pltpu.sync_copy(tposb, tlist2.at[pl.ds(wid * LCAP, LCAP)])


_scan = pl.kernel(
    _scan_body,
    out_type=(
        jax.ShapeDtypeStruct((NW * (LCAP // 2), 128), jnp.float32),
        jax.ShapeDtypeStruct((NW * LCAP,), jnp.int32),
    ),
    mesh=_mesh,
    scratch_types=[
        pltpu.VMEM((B,), jnp.int32),
        pltpu.VMEM((64, SLABW), jnp.float32),
        pltpu.VMEM((64, SLABW), jnp.float32),
        pltpu.VMEM((LCAP // 2, 128), jnp.float32),
        pltpu.VMEM((LCAP,), jnp.int32),
        pltpu.VMEM((LCAP,), jnp.int32),
        pltpu.VMEM((SUBN * SUBCAP,), jnp.int32),
        pltpu.VMEM((SUBN * SUBCAP,), jnp.int32),
        pltpu.VMEM((GCAP,), jnp.int32),
        pltpu.VMEM((GCAP,), jnp.int32),
        pltpu.SemaphoreType.DMA,
        pltpu.SemaphoreType.DMA,
    ],
    compiler_params=pltpu.CompilerParams(
        use_tc_tiling_on_sc=True, needs_layout_passes=False),
)


def _finish_body(rowsL, tlist2, pos_hbm, out_hbm, tl_v, tl2_v, tm_v, rows_v,
                 sem):
    wid = lax.axis_index("s") * 2 + lax.axis_index("c")
    pltpu.sync_copy(tlist2.at[pl.ds(wid * LCAP, LCAP)], tl_v)
    for j in range(8):
        pltpu.sync_copy(
            tlist2.at[pl.ds(wid * LCAP + j * 128, 128)], tl2_v.at[j])
    pltpu.sync_copy(rowsL.at[pl.ds(wid * LCAP, LCAP)], rows_v)

    def tmod(v, _):
        tm_v[pl.ds(v * 16, 16)] = tl_v[pl.ds(v * 16, 16)] & (SEQ - 1)
        return 0

    lax.fori_loop(0, LCAP // 16, tmod, 0)

    adds = [
        pltpu.async_copy(
            pos_hbm.at[tm_v.at[pl.ds(j * 128, 128)]],
            rows_v.at[pl.ds(j * 128, 128)],
            sem,
            add=True,
        )
        for j in range(8)
    ]
    for cp in adds:
        cp.wait()
    outs = [
        pltpu.async_copy(
            rows_v.at[pl.ds(j * 128, 128)], out_hbm.at[tl2_v.at[j]], sem)
        for j in range(8)
    ]
    for cp in outs:
        cp.wait()


_finish = pl.kernel(
    _finish_body,
    out_type=jax.ShapeDtypeStruct((OUTPAD, EMBED_DIM), jnp.float32),
    mesh=_mesh,
    scratch_types=[
        pltpu.VMEM((LCAP,), jnp.int32),
        pltpu.VMEM((8, 128), jnp.int32),
        pltpu.VMEM((LCAP,), jnp.int32),
        pltpu.VMEM((LCAP, EMBED_DIM), jnp.float32),
        pltpu.SemaphoreType.DMA,
    ],
    compiler_params=pltpu.CompilerParams(use_tc_tiling_on_sc=False),
)


@jax.jit
def kernel(x, table):
    tT = table.T
    xr = x.reshape(B).astype(jnp.int32)
    tailp = jnp.pad(tT[:, TAIL0:], ((0, 0), (0, SLABW - (VOCAB - TAIL0))))
    rows1, tlist2 = _scan(tT, xr, tailp)
    rowsL = rows1.reshape(NW * LCAP, EMBED_DIM)
    out = _finish(rowsL, tlist2, jnp.asarray(_POS_NP))
    return out[:B].reshape(BATCH, SEQ, EMBED_DIM)


# speedup vs baseline: 1.8053x; 1.8053x over previous
import functools

import jax
import jax.numpy as jnp
import numpy as np
from jax import lax
from jax.experimental import pallas as pl
from jax.experimental.pallas import tpu as pltpu
from jax.experimental.pallas import tpu_sc as plsc

EMBED_DIM = 64
BATCH = 4
SEQ = 4096
B = BATCH * SEQ
VOCAB = 1000000
NW = 32
RANGE = VOCAB // NW
SLABW = 256
NSLAB = 124
LCAP = 1024
SUBN = 8
SUBCAP = 256
GCAP = 256
LASTC = (VOCAB // SLABW) * SLABW - SLABW
TAIL0 = LASTC + SLABW
DUMP = B
OUTPAD = B + 128


def _positional_encoding_np(position, d_model):
    pos = np.arange(position)[:, np.newaxis].astype(np.float64)
    i = np.arange(d_model)[np.newaxis, :].astype(np.float64)
    angle_rates = 1.0 / np.power(10000, 2 * (i // 2) / np.float32(d_model))
    angle_rads = pos * angle_rates
    angle_rads[:, 0::2] = np.sin(angle_rads[:, 0::2])
    angle_rads[:, 1::2] = np.cos(angle_rads[:, 1::2])
    return angle_rads.astype(np.float32)


_POS_NP = _positional_encoding_np(SEQ, EMBED_DIM)

_mesh = plsc.VectorSubcoreMesh(core_axis_name="c", subcore_axis_name="s")
_IOTA = functools.partial(lax.broadcasted_iota, jnp.int32, (16,))


def _scan_body(tT, xr, tailp, rows1, tlist2, x_v, slab0, slab1, rows_v, idsb,
               tposb, subid, sublp, gcol, glp, sem0, sem1):
    wid = lax.axis_index("s") * 2 + lax.axis_index("c")
    lo = wid * RANGE
    c0 = (lo // SLABW) * SLABW
    iota = _IOTA(0)

    def slab_ds(start):
        return pl.ds(pl.multiple_of(jnp.minimum(start, LASTC), SLABW), SLABW)

    def slab_fetch(start, buf, sem):
        for r in range(8):
            pltpu.async_copy(
                tT.at[pl.ds(r * 8, 8), slab_ds(start)],
                buf.at[pl.ds(r * 8, 8)], sem)

    def slab_wait(buf, sem):
        for r in range(8):
            pltpu.make_async_copy(
                tT.at[pl.ds(0, 8), pl.ds(0, SLABW)],
                buf.at[pl.ds(r * 8, 8)], sem).wait()

    pltpu.sync_copy(xr, x_v)

    def init(v, _):
        tposb[pl.ds(v * 16, 16)] = jnp.zeros((16,), jnp.int32) + DUMP
        return 0

    lax.fori_loop(0, LCAP // 16, init, 0)

    slab_fetch(c0, slab0, sem0)
    slab_fetch(c0 + SLABW, slab1, sem1)

    def pref(v, n):
        ids = x_v[pl.ds(v * 16, 16)]
        m = (ids >= lo) & (ids < lo + RANGE)
        cnt = jnp.sum(jnp.where(m, 1, 0))
        nw = jnp.minimum(n, LCAP - 16)
        plsc.store_compressed(idsb.at[pl.ds(nw, 16)], ids, mask=m)
        plsc.store_compressed(tposb.at[pl.ds(nw, 16)], v * 16 + iota, mask=m)
        return n + cnt

    n = lax.fori_loop(0, B // 16, pref, jnp.zeros((), jnp.int32))

    def binstep(v, ks):
        ids = idsb[pl.ds(v * 16, 16)]
        lp = v * 16 + iota
        valid = lp < n
        sb = (ids - lo) >> 12
        out = []
        for s in range(SUBN):
            m = valid & (sb == s)
            cnt = jnp.sum(jnp.where(m, 1, 0))
            kw = jnp.minimum(ks[s], SUBCAP - 16)
            plsc.store_compressed(subid.at[pl.ds(s * SUBCAP + kw, 16)], ids, mask=m)
            plsc.store_compressed(sublp.at[pl.ds(s * SUBCAP + kw, 16)], lp, mask=m)
            out.append(ks[s] + cnt)
        return tuple(out)

    z = jnp.zeros((), jnp.int32)
    ks = lax.fori_loop(0, LCAP // 16, binstep, (z,) * SUBN)
    ns_vec = jnp.zeros((16,), jnp.int32)
    for s in range(SUBN):
        ns_vec = jnp.where(iota == s, ks[s], ns_vec)

    def sub_scan(s, start, k):
        ksub = jnp.sum(jnp.where(iota == s, ns_vec, 0))

        def scan(v, k):
            ids = subid[pl.ds(s * SUBCAP + v * 16, 16)]
            lp = sublp[pl.ds(s * SUBCAP + v * 16, 16)]
            valid = (v * 16 + iota) < ksub
            m = valid & (ids >= start) & (ids < start + SLABW)
            cnt = jnp.sum(jnp.where(m, 1, 0))
            kw = jnp.minimum(k, GCAP - 16)
            plsc.store_compressed(gcol.at[pl.ds(kw, 16)], ids - start, mask=m)
            plsc.store_compressed(glp.at[pl.ds(kw, 16)], lp, mask=m)
            return k + cnt

        return lax.fori_loop(0, (ksub + 15) >> 4, scan, k)

    def process(buf, start):
        sa = jnp.clip((start - lo) >> 12, 0, SUBN - 1)
        sb_ = jnp.clip((start + SLABW - 1 - lo) >> 12, 0, SUBN - 1)
        k = sub_scan(sa, start, jnp.zeros((), jnp.int32))
        k = sub_scan(jnp.minimum(jnp.maximum(sb_, sa + 1), SUBN - 1), start, k)

        def extract(g, _):
            msk = (g * 16 + iota) < k
            colsv = gcol[pl.ds(g * 16, 16)]
            lpv = glp[pl.ds(g * 16, 16)]
            for dd in range(EMBED_DIM):
                vals = plsc.load_gather(
                    buf, [jnp.zeros((16,), jnp.int32) + dd, colsv], mask=msk)
                plsc.store_scatter(
                    rows_v, [lpv >> 1, (lpv & 1) * 64 + dd], vals, mask=msk)
            return 0

        lax.fori_loop(0, (k + 15) >> 4, extract, 0)

    def slabloop(p, _):
        for b, (buf, sem) in enumerate(((slab0, sem0), (slab1, sem1))):
            ci = p * 2 + b
            start = jnp.minimum(c0 + ci * SLABW, LASTC)
            slab_wait(buf, sem)
            process(buf, start)
            slab_fetch(c0 + (ci + 2) * SLABW, buf, sem)
        return 0

    lax.fori_loop(0, NSLAB // 2, slabloop, 0)
    slab_wait(slab0, sem0)
    slab_wait(slab1, sem1)

    @pl.when(wid == NW - 1)
    def _():
        pltpu.sync_copy(tailp, slab0)
        process(slab0, jnp.zeros((), jnp.int32) + TAIL0)

    pltpu.sync_copy(rows_v, rows1.at[pl.ds(wid * (LCAP // 2), LCAP // 2)])
    pltpu.sync_copy(tposb, tlist2.at[pl.ds(wid * LCAP, LCAP)])


_scan = pl.kernel(
    _scan_body,
    out_type=(
        jax.ShapeDtypeStruct((NW * (LCAP // 2), 128), jnp.float32),
        jax.ShapeDtypeStruct((NW * LCAP,), jnp.int32),
    ),
    mesh=_mesh,
    scratch_types=[
        pltpu.VMEM((B,), jnp.int32),
        pltpu.VMEM((64, SLABW), jnp.float32),
        pltpu.VMEM((64, SLABW), jnp.float32),
        pltpu.VMEM((LCAP // 2, 128), jnp.float32),
        pltpu.VMEM((LCAP,), jnp.int32),
        pltpu.VMEM((LCAP,), jnp.int32),
        pltpu.VMEM((SUBN * SUBCAP,), jnp.int32),
        pltpu.VMEM((SUBN * SUBCAP,), jnp.int32),
        pltpu.VMEM((GCAP,), jnp.int32),
        pltpu.VMEM((GCAP,), jnp.int32),
        pltpu.SemaphoreType.DMA,
        pltpu.SemaphoreType.DMA,
    ],
    compiler_params=pltpu.CompilerParams(
        use_tc_tiling_on_sc=True, needs_layout_passes=False),
)


def _finish_body(rowsL, tlist2, pos_hbm, out_hbm, tl_v, tl2_v, tm_v, rows_v,
                 sem):
    wid = lax.axis_index("s") * 2 + lax.axis_index("c")
    pltpu.sync_copy(tlist2.at[pl.ds(wid * LCAP, LCAP)], tl_v)
    for j in range(8):
        pltpu.sync_copy(
            tlist2.at[pl.ds(wid * LCAP + j * 128, 128)], tl2_v.at[j])
    pltpu.sync_copy(rowsL.at[pl.ds(wid * LCAP, LCAP)], rows_v)

    def tmod(v, _):
        tm_v[pl.ds(v * 16, 16)] = tl_v[pl.ds(v * 16, 16)] & (SEQ - 1)
        return 0

    lax.fori_loop(0, LCAP // 16, tmod, 0)

    adds = [
        pltpu.async_copy(
            pos_hbm.at[tm_v.at[pl.ds(j * 128, 128)]],
            rows_v.at[pl.ds(j * 128, 128)],
            sem,
            add=True,
        )
        for j in range(8)
    ]
    for cp in adds:
        cp.wait()
    outs = [
        pltpu.async_copy(
            rows_v.at[pl.ds(j * 128, 128)], out_hbm.at[tl2_v.at[j]], sem)
        for j in range(8)
    ]
    for cp in outs:
        cp.wait()


_finish = pl.kernel(
    _finish_body,
    out_type=jax.ShapeDtypeStruct((OUTPAD, EMBED_DIM), jnp.float32),
    mesh=_mesh,
    scratch_types=[
        pltpu.VMEM((LCAP,), jnp.int32),
        pltpu.VMEM((8, 128), jnp.int32),
        pltpu.VMEM((LCAP,), jnp.int32),
        pltpu.VMEM((LCAP, EMBED_DIM), jnp.float32),
        pltpu.SemaphoreType.DMA,
    ],
    compiler_params=pltpu.CompilerParams(use_tc_tiling_on_sc=False),
)


@jax.jit
def kernel(x, table):
    tT = table.T
    xr = x.reshape(B).astype(jnp.int32)
    tailp = jnp.pad(tT[:, TAIL0:], ((0, 0), (0, SLABW - (VOCAB - TAIL0))))
    rows1, tlist2 = _scan(tT, xr, tailp)
    rowsL = rows1.reshape(NW * LCAP, EMBED_DIM)
    out = _finish(rowsL, tlist2, jnp.asarray(_POS_NP))
    return out[:B].reshape(BATCH, SEQ, EMBED_DIM)


# speedup vs baseline: 6.7382x; 3.7326x over previous
import functools

import jax
import jax.numpy as jnp
import numpy as np
from jax import lax
from jax.experimental import pallas as pl
from jax.experimental.pallas import tpu as pltpu
from jax.experimental.pallas import tpu_sc as plsc

EMBED_DIM = 64
BATCH = 4
SEQ = 4096
B = BATCH * SEQ
VOCAB = 1000000
NW = 32
RANGE = VOCAB // NW
SLABW = 128
NSLAB = 248
LCAP = 1024
SUBN = 8
SUBCAP = 256
GCAP = 256
LASTC = (VOCAB // SLABW) * SLABW - SLABW
TAIL0 = LASTC + SLABW
DUMP = B
OUTPAD = B + 128


def _positional_encoding_np(position, d_model):
    pos = np.arange(position)[:, np.newaxis].astype(np.float64)
    i = np.arange(d_model)[np.newaxis, :].astype(np.float64)
    angle_rates = 1.0 / np.power(10000, 2 * (i // 2) / np.float32(d_model))
    angle_rads = pos * angle_rates
    angle_rads[:, 0::2] = np.sin(angle_rads[:, 0::2])
    angle_rads[:, 1::2] = np.cos(angle_rads[:, 1::2])
    return angle_rads.astype(np.float32)


_POS_NP = _positional_encoding_np(SEQ, EMBED_DIM)

_mesh = plsc.VectorSubcoreMesh(core_axis_name="c", subcore_axis_name="s")
_IOTA = functools.partial(lax.broadcasted_iota, jnp.int32, (16,))


def _scan_body(tT, xr, tailp, rows1, tlist2, x_v, slab0, slab1, slab2, slab3,
               rows_v, idsb, tposb, subid, sublp, gcol, glp, sem0, sem1, sem2,
               sem3):
    wid = lax.axis_index("s") * 2 + lax.axis_index("c")
    lo = wid * RANGE
    c0 = (lo // SLABW) * SLABW
    iota = _IOTA(0)

    def slab_ds(start):
        return pl.ds(pl.multiple_of(jnp.minimum(start, LASTC), SLABW), SLABW)

    def slab_fetch(start, buf, sem):
        for r in range(8):
            pltpu.async_copy(
                tT.at[pl.ds(r * 8, 8), slab_ds(start)],
                buf.at[pl.ds(r * 8, 8)], sem)

    def slab_wait(buf, sem):
        for r in range(8):
            pltpu.make_async_copy(
                tT.at[pl.ds(0, 8), pl.ds(0, SLABW)],
                buf.at[pl.ds(r * 8, 8)], sem).wait()

    pltpu.sync_copy(xr, x_v)

    def init(v, _):
        tposb[pl.ds(v * 16, 16)] = DUMP + wid * 4 + (iota & 3)
        return 0

    lax.fori_loop(0, LCAP // 16, init, 0)

    rings = ((slab0, sem0), (slab1, sem1), (slab2, sem2), (slab3, sem3))
    for b, (buf, sem) in enumerate(rings):
        slab_fetch(c0 + b * SLABW, buf, sem)

    def pref(v, n):
        ids = x_v[pl.ds(v * 16, 16)]
        m = (ids >= lo) & (ids < lo + RANGE)
        cnt = jnp.sum(jnp.where(m, 1, 0))
        nw = jnp.minimum(n, LCAP - 16)
        plsc.store_compressed(idsb.at[pl.ds(nw, 16)], ids, mask=m)
        plsc.store_compressed(tposb.at[pl.ds(nw, 16)], v * 16 + iota, mask=m)
        return n + cnt

    n = lax.fori_loop(0, B // 16, pref, jnp.zeros((), jnp.int32))

    def binstep(v, ks):
        ids = idsb[pl.ds(v * 16, 16)]
        lp = v * 16 + iota
        valid = lp < n
        sb = (ids - lo) >> 12
        out = []
        for s in range(SUBN):
            m = valid & (sb == s)
            cnt = jnp.sum(jnp.where(m, 1, 0))
            kw = jnp.minimum(ks[s], SUBCAP - 16)
            plsc.store_compressed(subid.at[pl.ds(s * SUBCAP + kw, 16)], ids, mask=m)
            plsc.store_compressed(sublp.at[pl.ds(s * SUBCAP + kw, 16)], lp, mask=m)
            out.append(ks[s] + cnt)
        return tuple(out)

    z = jnp.zeros((), jnp.int32)
    ks = lax.fori_loop(0, LCAP // 16, binstep, (z,) * SUBN)
    ns_vec = jnp.zeros((16,), jnp.int32)
    for s in range(SUBN):
        ns_vec = jnp.where(iota == s, ks[s], ns_vec)

    def sub_scan(s, start, k):
        ksub = jnp.sum(jnp.where(iota == s, ns_vec, 0))

        def scan(v, k):
            ids = subid[pl.ds(s * SUBCAP + v * 16, 16)]
            lp = sublp[pl.ds(s * SUBCAP + v * 16, 16)]
            valid = (v * 16 + iota) < ksub
            m = valid & (ids >= start) & (ids < start + SLABW)
            cnt = jnp.sum(jnp.where(m, 1, 0))
            kw = jnp.minimum(k, GCAP - 16)
            plsc.store_compressed(gcol.at[pl.ds(kw, 16)], ids - start, mask=m)
            plsc.store_compressed(glp.at[pl.ds(kw, 16)], lp, mask=m)
            return k + cnt

        return lax.fori_loop(0, (ksub + 15) >> 4, scan, k)

    def process(buf, start):
        sa = jnp.clip((start - lo) >> 12, 0, SUBN - 1)
        sb_ = jnp.clip((start + SLABW - 1 - lo) >> 12, 0, SUBN - 1)
        k = sub_scan(sa, start, jnp.zeros((), jnp.int32))
        k = sub_scan(jnp.minimum(jnp.maximum(sb_, sa + 1), SUBN - 1), start, k)

        def extract(g, _):
            msk = (g * 16 + iota) < k
            colsv = gcol[pl.ds(g * 16, 16)]
            lpv = glp[pl.ds(g * 16, 16)]
            for dd in range(EMBED_DIM):
                vals = plsc.load_gather(
                    buf, [jnp.zeros((16,), jnp.int32) + dd, colsv], mask=msk)
                plsc.store_scatter(
                    rows_v, [lpv >> 1, (lpv & 1) * 64 + dd], vals, mask=msk)
            return 0

        lax.fori_loop(0, (k + 15) >> 4, extract, 0)

    def slabloop(p, _):
        for b, (buf, sem) in enumerate(rings):
            ci = p * 4 + b
            start = jnp.minimum(c0 + ci * SLABW, LASTC)
            slab_wait(buf, sem)
            process(buf, start)
            slab_fetch(c0 + (ci + 4) * SLABW, buf, sem)
        return 0

    lax.fori_loop(0, NSLAB // 4, slabloop, 0)
    for buf, sem in rings:
        slab_wait(buf, sem)

    @pl.when(wid == NW - 1)
    def _():
        pltpu.sync_copy(tailp, slab0)
        process(slab0, jnp.zeros((), jnp.int32) + TAIL0)

    pltpu.sync_copy(rows_v, rows1.at[pl.ds(wid * (LCAP // 2), LCAP // 2)])
    pltpu.sync_copy(tposb, tlist2.at[pl.ds(wid * LCAP, LCAP)])


_scan = pl.kernel(
    _scan_body,
    out_type=(
        jax.ShapeDtypeStruct((NW * (LCAP // 2), 128), jnp.float32),
        jax.ShapeDtypeStruct((NW * LCAP,), jnp.int32),
    ),
    mesh=_mesh,
    scratch_types=[
        pltpu.VMEM((B,), jnp.int32),
        pltpu.VMEM((64, SLABW), jnp.float32),
        pltpu.VMEM((64, SLABW), jnp.float32),
        pltpu.VMEM((64, SLABW), jnp.float32),
        pltpu.VMEM((64, SLABW), jnp.float32),
        pltpu.VMEM((LCAP // 2, 128), jnp.float32),
        pltpu.VMEM((LCAP,), jnp.int32),
        pltpu.VMEM((LCAP,), jnp.int32),
        pltpu.VMEM((SUBN * SUBCAP,), jnp.int32),
        pltpu.VMEM((SUBN * SUBCAP,), jnp.int32),
        pltpu.VMEM((GCAP,), jnp.int32),
        pltpu.VMEM((GCAP,), jnp.int32),
        pltpu.SemaphoreType.DMA,
        pltpu.SemaphoreType.DMA,
        pltpu.SemaphoreType.DMA,
        pltpu.SemaphoreType.DMA,
    ],
    compiler_params=pltpu.CompilerParams(
        use_tc_tiling_on_sc=True, needs_layout_passes=False),
)


def _finish_body(rowsL, tlist2, pos_hbm, out_hbm, tl_v, tl2_v, tm_v, rows_v,
                 sem):
    wid = lax.axis_index("s") * 2 + lax.axis_index("c")
    pltpu.sync_copy(tlist2.at[pl.ds(wid * LCAP, LCAP)], tl_v)
    for j in range(8):
        pltpu.sync_copy(
            tlist2.at[pl.ds(wid * LCAP + j * 128, 128)], tl2_v.at[j])
    pltpu.sync_copy(rowsL.at[pl.ds(wid * LCAP, LCAP)], rows_v)

    def tmod(v, _):
        tm_v[pl.ds(v * 16, 16)] = tl_v[pl.ds(v * 16, 16)] & (SEQ - 1)
        return 0

    lax.fori_loop(0, LCAP // 16, tmod, 0)

    adds = [
        pltpu.async_copy(
            pos_hbm.at[tm_v.at[pl.ds(j * 128, 128)]],
            rows_v.at[pl.ds(j * 128, 128)],
            sem,
            add=True,
        )
        for j in range(8)
    ]
    for cp in adds:
        cp.wait()
    outs = [
        pltpu.async_copy(
            rows_v.at[pl.ds(j * 128, 128)], out_hbm.at[tl2_v.at[j]], sem)
        for j in range(8)
    ]
    for cp in outs:
        cp.wait()


_finish = pl.kernel(
    _finish_body,
    out_type=jax.ShapeDtypeStruct((OUTPAD, EMBED_DIM), jnp.float32),
    mesh=_mesh,
    scratch_types=[
        pltpu.VMEM((LCAP,), jnp.int32),
        pltpu.VMEM((8, 128), jnp.int32),
        pltpu.VMEM((LCAP,), jnp.int32),
        pltpu.VMEM((LCAP, EMBED_DIM), jnp.float32),
        pltpu.SemaphoreType.DMA,
    ],
    compiler_params=pltpu.CompilerParams(use_tc_tiling_on_sc=False),
)


@jax.jit
def kernel(x, table):
    tT = table.T
    xr = x.reshape(B).astype(jnp.int32)
    tailp = jnp.pad(tT[:, TAIL0:], ((0, 0), (0, SLABW - (VOCAB - TAIL0))))
    rows1, tlist2 = _scan(tT, xr, tailp)
    rowsL = rows1.reshape(NW * LCAP, EMBED_DIM)
    out = _finish(rowsL, tlist2, jnp.asarray(_POS_NP))
    return out[:B].reshape(BATCH, SEQ, EMBED_DIM)
